# Initial kernel scaffold; baseline (speedup 1.0000x reference)
import jax
import jax.numpy as jnp
from jax.experimental import pallas as pl


def kernel(embed_user_weight, embed_item_weight, edge_user, edge_item, edge_vals, user, item_i, item_j):
    raise NotImplementedError("write your pallas kernel here")


# speedup vs baseline: 2.7674x; 2.7674x over previous
import functools

import jax
import jax.numpy as jnp
from jax import lax
from jax.experimental import pallas as pl
from jax.experimental.pallas import tpu as pltpu
from jax.experimental.pallas import tpu_sc as plsc

D = 64
NC = 2
NS = 16
L = 16
BLK = 128
SB = 8
EDGE_TILE = BLK * SB
CH = 32
ROW_ALIGN = NS * CH
BIG = 1 << 30


def _rup(x, m):
    return ((x + m - 1) // m) * m


def _iota16():
    return lax.broadcasted_iota(jnp.int32, (L,), 0)


def _make_spmm(n_out, nnz_pad):
    half0 = _rup((n_out + 1) // 2, ROW_ALIGN)
    assert half0 < n_out
    n1 = n_out - half0
    copy0 = half0
    copy1 = _rup(n1, ROW_ALIGN)
    out_pad = half0 + copy1
    acc_copy = max(copy0, copy1)
    dump = acc_copy
    acc_rows = acc_copy + 8
    assert nnz_pad % (NS * EDGE_TILE) == 0
    ept = nnz_pad // NS
    n_sb = ept // EDGE_TILE

    mesh = plsc.VectorSubcoreMesh(core_axis_name="c", subcore_axis_name="s")

    @functools.partial(
        pl.kernel,
        out_type=jax.ShapeDtypeStruct((out_pad, D), jnp.float32),
        mesh=mesh,
        compiler_params=pltpu.CompilerParams(use_tc_tiling_on_sc=False),
        scratch_types=[
            pltpu.VMEM_SHARED((acc_rows, D), jnp.float32),
            pltpu.VMEM((EDGE_TILE,), jnp.int32),
            pltpu.VMEM((EDGE_TILE,), jnp.int32),
            pltpu.VMEM((EDGE_TILE,), jnp.float32),
            pltpu.VMEM((BLK, D), jnp.float32),
            pltpu.VMEM((BLK,), jnp.int32),
            pltpu.SemaphoreType.DMA,
        ],
    )
    def spmm(x_hbm, rows_hbm, cols_hbm, vals_hbm, out_hbm,
             acc, rows_v, cols_v, vals_v, gbuf, sidx, gsem):
        c = lax.axis_index("c")
        s = lax.axis_index("s")
        lo = c * half0
        hi = jnp.where(c == 0, half0, n_out)
        copy_rows = jnp.where(c == 0, copy0, copy1)
        zeros16 = jnp.zeros((L,), jnp.float32)
        iota = _iota16()

        for r in range(CH):
            for q in range(D // L):
                gbuf[r, pl.ds(q * L, L)] = zeros16

        def zloop(k, _):
            r0 = s * (acc_copy // NS) + k * CH
            pltpu.sync_copy(gbuf.at[pl.ds(0, CH)], acc.at[pl.ds(r0, CH)])
            return 0

        lax.fori_loop(0, acc_copy // (NS * CH), zloop, 0)
        plsc.subcore_barrier()

        tile_base = s * ept

        def sb_loop(t, _):
            mb = tile_base + t * EDGE_TILE
            pltpu.sync_copy(rows_hbm.at[pl.ds(mb, EDGE_TILE)], rows_v)
            pltpu.sync_copy(cols_hbm.at[pl.ds(mb, EDGE_TILE)], cols_v)
            pltpu.sync_copy(vals_hbm.at[pl.ds(mb, EDGE_TILE)], vals_v)

            def blk_loop(b, _):
                cp = pltpu.async_copy(
                    x_hbm.at[cols_v.at[pl.ds(b * BLK, BLK)]], gbuf, gsem)

                def grp(g, _):
                    r16 = rows_v[pl.ds(b * BLK + g * L, L)]
                    m = (r16 >= lo) & (r16 < hi)
                    sidx[pl.ds(g * L, L)] = jnp.where(m, r16 - lo, dump)
                    return 0

                lax.fori_loop(0, BLK // L, grp, 0)
                cp.wait()

                for g in range(BLK // L):
                    vv = vals_v[pl.ds(b * BLK + g * L, L)]
                    for k in range(L):
                        e = g * L + k
                        v16 = jnp.full((L,), vv[k])
                        for q in range(D // L):
                            sl = pl.ds(q * L, L)
                            gbuf[e, sl] = gbuf[e, sl] * v16
                pltpu.sync_copy(gbuf, acc.at[sidx], add=True)
                return 0

            lax.fori_loop(0, SB, blk_loop, 0)
            return 0

        lax.fori_loop(0, n_sb, sb_loop, 0)
        plsc.subcore_barrier()

        def oloop(k, _):
            r0 = s * CH + k * (NS * CH)
            pltpu.sync_copy(acc.at[pl.ds(r0, CH)], gbuf.at[pl.ds(0, CH)])
            pltpu.sync_copy(gbuf.at[pl.ds(0, CH)], out_hbm.at[pl.ds(lo + r0, CH)])
            return 0

        lax.fori_loop(0, copy_rows // (NS * CH), oloop, 0)

    return spmm


def _make_combine_gather(n_tab_pad, b_pad):
    assert b_pad % (NC * NS * BLK) == 0
    per_tile = b_pad // (NC * NS)
    n_blk = per_tile // BLK
    mesh = plsc.VectorSubcoreMesh(core_axis_name="c", subcore_axis_name="s")

    @functools.partial(
        pl.kernel,
        out_type=jax.ShapeDtypeStruct((b_pad, D), jnp.float32),
        mesh=mesh,
        compiler_params=pltpu.CompilerParams(use_tc_tiling_on_sc=False),
        scratch_types=[
            pltpu.VMEM((per_tile,), jnp.int32),
            pltpu.VMEM((BLK, D), jnp.float32),
            pltpu.VMEM((BLK, D), jnp.float32),
            pltpu.VMEM((BLK, D), jnp.float32),
            pltpu.VMEM((BLK, D), jnp.float32),
            pltpu.SemaphoreType.DMA,
        ],
    )
    def comb(t0_hbm, t1_hbm, t2_hbm, t3_hbm, idx_hbm, out_hbm,
             idx_v, b0, b1, b2, b3, sem):
        c = lax.axis_index("c")
        s = lax.axis_index("s")
        wid = s * NC + c
        base = wid * per_tile
        pltpu.sync_copy(idx_hbm.at[pl.ds(base, per_tile)], idx_v)
        iota = _iota16()

        def blk_loop(k, _):
            ids = idx_v.at[pl.ds(k * BLK, BLK)]
            pltpu.async_copy(t0_hbm.at[ids], b0, sem).wait()
            pltpu.async_copy(t1_hbm.at[ids], b1, sem).wait()
            pltpu.async_copy(t2_hbm.at[ids], b2, sem).wait()
            pltpu.async_copy(t3_hbm.at[ids], b3, sem).wait()

            for r in range(BLK):
                for q in range(D // L):
                    sl = pl.ds(q * L, L)
                    b0[r, sl] = (b0[r, sl] + 0.5 * b1[r, sl]
                                 + (1.0 / 3.0) * b2[r, sl]
                                 + 0.25 * b3[r, sl])
            pltpu.sync_copy(b0, out_hbm.at[pl.ds(base + k * BLK, BLK)])
            return 0

        lax.fori_loop(0, n_blk, blk_loop, 0)

    return comb


def _tc_finish(u, i, j):
    b = u.shape[0]

    def body(u_ref, i_ref, j_ref, pi_ref, pj_ref, loss_ref, loss2_ref):
        uu = u_ref[...]
        ii = i_ref[...]
        jj = j_ref[...]
        pi = jnp.sum(uu * ii, axis=1, keepdims=True)
        pj = jnp.sum(uu * jj, axis=1, keepdims=True)
        d = pi - pj
        bpr = jnp.mean(jnp.maximum(-d, 0.0) + jnp.log1p(jnp.exp(-jnp.abs(d))))
        l2 = 1e-4 * jnp.sum(uu * uu + ii * ii + jj * jj, axis=1, keepdims=True)
        pi_ref[...] = pi
        pj_ref[...] = pj
        loss_ref[0, 0] = bpr + jnp.mean(l2)
        loss2_ref[0, 0] = bpr

    return pl.pallas_call(
        body,
        out_shape=(
            jax.ShapeDtypeStruct((b, 1), jnp.float32),
            jax.ShapeDtypeStruct((b, 1), jnp.float32),
            jax.ShapeDtypeStruct((1, 1), jnp.float32),
            jax.ShapeDtypeStruct((1, 1), jnp.float32),
        ),
        out_specs=(
            pl.BlockSpec(memory_space=pltpu.VMEM),
            pl.BlockSpec(memory_space=pltpu.VMEM),
            pl.BlockSpec(memory_space=pltpu.SMEM),
            pl.BlockSpec(memory_space=pltpu.SMEM),
        ),
    )(u, i, j)


def kernel(embed_user_weight, embed_item_weight, edge_user, edge_item,
           edge_vals, user, item_i, item_j):
    n_user = embed_user_weight.shape[0]
    n_item = embed_item_weight.shape[0]
    nnz = edge_user.shape[0]
    batch = user.shape[0]

    nnz_pad = _rup(nnz, NS * EDGE_TILE)
    pad = nnz_pad - nnz
    rows_u = jnp.concatenate([edge_user, jnp.full((pad,), BIG, jnp.int32)])
    rows_i = jnp.concatenate([edge_item, jnp.full((pad,), BIG, jnp.int32)])
    cols_u = jnp.concatenate([edge_item, jnp.zeros((pad,), jnp.int32)])
    cols_i = jnp.concatenate([edge_user, jnp.zeros((pad,), jnp.int32)])
    vals_p = jnp.concatenate([edge_vals, jnp.zeros((pad,), jnp.float32)])

    spmm_u = _make_spmm(n_user, nnz_pad)
    spmm_i = _make_spmm(n_item, nnz_pad)

    upad = _rup((n_user + 1) // 2, ROW_ALIGN)
    upad = upad + _rup(n_user - upad, ROW_ALIGN)
    ipad = _rup((n_item + 1) // 2, ROW_ALIGN)
    ipad = ipad + _rup(n_item - ipad, ROW_ALIGN)
    eu = jnp.concatenate(
        [embed_user_weight,
         jnp.zeros((upad - n_user, D), jnp.float32)])
    ei = jnp.concatenate(
        [embed_item_weight,
         jnp.zeros((ipad - n_item, D), jnp.float32)])

    g1u = spmm_u(ei, rows_u, cols_u, vals_p)
    g1i = spmm_i(eu, rows_i, cols_i, vals_p)
    g2u = spmm_u(g1i, rows_u, cols_u, vals_p)
    g2i = spmm_i(g1u, rows_i, cols_i, vals_p)
    g3u = spmm_u(g2i, rows_u, cols_u, vals_p)
    g3i = spmm_i(g2u, rows_i, cols_i, vals_p)

    b_pad = _rup(batch, NC * NS * BLK)
    bp = b_pad - batch
    uidx = jnp.concatenate([user, jnp.zeros((bp,), jnp.int32)])
    iidx = jnp.concatenate([item_i, jnp.zeros((bp,), jnp.int32)])
    jidx = jnp.concatenate([item_j, jnp.zeros((bp,), jnp.int32)])

    comb_u = _make_combine_gather(upad, b_pad)
    comb_i = _make_combine_gather(ipad, b_pad)
    u_rows = comb_u(eu, g1u, g2u, g3u, uidx)
    i_rows = comb_i(ei, g1i, g2i, g3i, iidx)
    j_rows = comb_i(ei, g1i, g2i, g3i, jidx)

    pi, pj, loss, loss_ = _tc_finish(
        u_rows[:batch], i_rows[:batch], j_rows[:batch])
    return (pi[:, 0], pj[:, 0], loss[0, 0], loss_[0, 0])
